# baseline (speedup 1.0000x reference)
import jax
import jax.numpy as jnp
from jax.experimental import pallas as pl
from jax.experimental.pallas import tpu as pltpu

_D_MODEL = 2048
_N_EXP = 16
_T = 16384
_TM = 512


def _router_block(h_ref, wt_ref, b_ref, mask_ref, probs_ref, logits_ref):
    logits = jnp.dot(h_ref[...], wt_ref[...], preferred_element_type=jnp.float32)
    lt = logits.T + b_ref[...]
    logits_ref[...] = lt

    m1 = jnp.max(lt, axis=0, keepdims=True)
    e = jnp.exp(lt - m1)
    probs_ref[...] = e / jnp.sum(e, axis=0, keepdims=True)

    row = jax.lax.broadcasted_iota(jnp.int32, lt.shape, 0).astype(jnp.float32)
    big = jnp.float32(_N_EXP)
    i1 = jnp.min(jnp.where(lt == m1, row, big), axis=0, keepdims=True)
    is1 = row == i1
    rest = jnp.where(is1, -jnp.inf, lt)
    m2 = jnp.max(rest, axis=0, keepdims=True)
    i2 = jnp.min(jnp.where(rest == m2, row, big), axis=0, keepdims=True)
    mask_ref[...] = (is1 | (row == i2)).astype(jnp.float32)


def kernel(h, bias_row, W):
    wt = W.T
    b = jnp.zeros((_N_EXP, 1), jnp.float32).at[_N_EXP - 1, 0].set(bias_row[-1])
    out_shapes = (
        jax.ShapeDtypeStruct((_N_EXP, _T), jnp.float32),
        jax.ShapeDtypeStruct((_N_EXP, _T), jnp.float32),
        jax.ShapeDtypeStruct((_N_EXP, _T), jnp.float32),
    )
    out_spec = pl.BlockSpec((_N_EXP, _TM), lambda i: (0, i))
    mask_f, probs, logits = pl.pallas_call(
        _router_block,
        grid=(_T // _TM,),
        in_specs=[
            pl.BlockSpec((_TM, _D_MODEL), lambda i: (i, 0)),
            pl.BlockSpec((_D_MODEL, _N_EXP), lambda i: (0, 0)),
            pl.BlockSpec((_N_EXP, 1), lambda i: (0, 0)),
        ],
        out_specs=(out_spec, out_spec, out_spec),
        out_shape=out_shapes,
        compiler_params=pltpu.CompilerParams(
            dimension_semantics=("arbitrary",),
        ),
    )(h, wt, b)
    return (mask_f.T.astype(bool), probs.T, logits.T)


# speedup vs baseline: 1.1673x; 1.1673x over previous
import jax
import jax.numpy as jnp
from jax.experimental import pallas as pl
from jax.experimental.pallas import tpu as pltpu

_D_MODEL = 2048
_N_EXP = 16
_T = 16384
_TM = 1024


def _router_block(h_ref, wt_ref, b_ref, mask_ref, probs_ref, logits_ref):
    logits = jnp.dot(h_ref[...], wt_ref[...], preferred_element_type=jnp.float32)
    lt = logits.T + b_ref[...]
    logits_ref[...] = lt

    m1 = jnp.max(lt, axis=0, keepdims=True)
    e = jnp.exp(lt - m1)
    probs_ref[...] = e / jnp.sum(e, axis=0, keepdims=True)

    row = jax.lax.broadcasted_iota(jnp.int32, lt.shape, 0).astype(jnp.float32)
    big = jnp.float32(_N_EXP)
    i1 = jnp.min(jnp.where(lt == m1, row, big), axis=0, keepdims=True)
    is1 = row == i1
    rest = jnp.where(is1, -jnp.inf, lt)
    m2 = jnp.max(rest, axis=0, keepdims=True)
    i2 = jnp.min(jnp.where(rest == m2, row, big), axis=0, keepdims=True)
    mask_ref[...] = (is1 | (row == i2)).astype(jnp.float32)


def kernel(h, bias_row, W):
    wt = W.T
    b = jnp.zeros((_N_EXP, 1), jnp.float32).at[_N_EXP - 1, 0].set(bias_row[-1])
    out_shapes = (
        jax.ShapeDtypeStruct((_N_EXP, _T), jnp.float32),
        jax.ShapeDtypeStruct((_N_EXP, _T), jnp.float32),
        jax.ShapeDtypeStruct((_N_EXP, _T), jnp.float32),
    )
    out_spec = pl.BlockSpec((_N_EXP, _TM), lambda i: (0, i))
    mask_f, probs, logits = pl.pallas_call(
        _router_block,
        grid=(_T // _TM,),
        in_specs=[
            pl.BlockSpec((_TM, _D_MODEL), lambda i: (i, 0)),
            pl.BlockSpec((_D_MODEL, _N_EXP), lambda i: (0, 0)),
            pl.BlockSpec((_N_EXP, 1), lambda i: (0, 0)),
        ],
        out_specs=(out_spec, out_spec, out_spec),
        out_shape=out_shapes,
        compiler_params=pltpu.CompilerParams(
            dimension_semantics=("arbitrary",),
        ),
    )(h, wt, b)
    return (mask_f.T.astype(bool), probs.T, logits.T)
